# baseline (speedup 1.0000x reference)
import functools

import jax
import jax.numpy as jnp
from jax import lax
from jax.experimental import pallas as pl
from jax.experimental.pallas import tpu as pltpu


def _fps_body(m, x_ref, y_ref, z_ref, out_ref):
    R = x_ref.shape[0]
    x = x_ref[:, :]
    y = y_ref[:, :]
    z = z_ref[:, :]
    rows_out = m // 128
    iota_flat = (lax.broadcasted_iota(jnp.int32, (R, 128), 0) * 128
                 + lax.broadcasted_iota(jnp.int32, (R, 128), 1))
    lane = lax.broadcasted_iota(jnp.int32, (1, 128), 1)
    out_iota = (lax.broadcasted_iota(jnp.int32, (rows_out, 128), 0) * 128
                + lax.broadcasted_iota(jnp.int32, (rows_out, 128), 1))

    def body(i, state):
        dists, idxs, r, c = state
        idxs = jnp.where(out_iota == i, r * 128 + c, idxs)
        cm = lane == c
        xf = jnp.sum(jnp.where(cm, x_ref[pl.ds(r, 1), :], 0.0))
        yf = jnp.sum(jnp.where(cm, y_ref[pl.ds(r, 1), :], 0.0))
        zf = jnp.sum(jnp.where(cm, z_ref[pl.ds(r, 1), :], 0.0))
        d = (x - xf) ** 2 + (y - yf) ** 2 + (z - zf) ** 2
        dists = jnp.minimum(dists, d)
        mx = jnp.max(dists)
        far = jnp.min(jnp.where(dists == mx, iota_flat, jnp.int32(R * 128)))
        return dists, idxs, far // 128, far % 128

    dists0 = jnp.full((R, 128), 1e10, dtype=jnp.float32)
    idxs0 = jnp.zeros((rows_out, 128), dtype=jnp.int32)
    _, idxs, _, _ = lax.fori_loop(
        0, m, body, (dists0, idxs0, jnp.int32(0), jnp.int32(0)))
    out_ref[:, :] = idxs


def _fps(pos, m):
    n = pos.shape[0]
    x = pos[:, 0].reshape(n // 128, 128)
    y = pos[:, 1].reshape(n // 128, 128)
    z = pos[:, 2].reshape(n // 128, 128)
    idxs = pl.pallas_call(
        functools.partial(_fps_body, m),
        out_shape=jax.ShapeDtypeStruct((m // 128, 128), jnp.int32),
    )(x, y, z)
    return idxs.reshape(m)


def _topk_body(n, k, kpad, chunk, a_ref, bt_ref, idx_ref, d_ref, dmat):
    BR = a_ref.shape[0]
    nchunks = n // chunk
    BIG = jnp.float32(3.0e38)
    lane_k = lax.broadcasted_iota(jnp.int32, (BR, kpad), 1)
    a = a_ref[:, :]
    a16 = a.astype(jnp.bfloat16)
    qq = jnp.sum(a[:, 0:3] * a[:, 0:3], axis=1, keepdims=True)

    def mmbody(c, _):
        bt = bt_ref[:, pl.ds(c * chunk, chunk)]
        p = jnp.dot(a16, bt.astype(jnp.bfloat16),
                    preferred_element_type=jnp.float32)
        rr = bt[3:4, :]
        dmat[:, pl.ds(c * chunk, chunk)] = (qq + rr) - 2.0 * p
        return 0

    lax.fori_loop(0, nchunks, mmbody, 0)

    def jbody(j, carry):
        idxs, dvs = carry

        def cbody(c, st):
            mn, am = st
            base = c * chunk
            v = dmat[:, pl.ds(base, chunk)]
            cmn = jnp.min(v, axis=1, keepdims=True)
            ci = lax.broadcasted_iota(jnp.int32, (BR, chunk), 1) + base
            cam = jnp.min(jnp.where(v == cmn, ci, jnp.int32(n)), axis=1,
                          keepdims=True)
            upd = cmn < mn
            return jnp.where(upd, cmn, mn), jnp.where(upd, cam, am)

        mn0 = jnp.full((BR, 1), BIG, jnp.float32)
        am0 = jnp.full((BR, 1), n, jnp.int32)
        mn, am = lax.fori_loop(0, nchunks, cbody, (mn0, am0))

        def mbody(c, _):
            base = c * chunk
            ci = lax.broadcasted_iota(jnp.int32, (BR, chunk), 1) + base
            cur = dmat[:, pl.ds(base, chunk)]
            dmat[:, pl.ds(base, chunk)] = jnp.where(ci == am, BIG, cur)
            return 0

        lax.fori_loop(0, nchunks, mbody, 0)
        idxs = jnp.where(lane_k == j, am, idxs)
        dvs = jnp.where(lane_k == j, mn, dvs)
        return idxs, dvs

    idxs0 = jnp.zeros((BR, kpad), jnp.int32)
    dvs0 = jnp.zeros((BR, kpad), jnp.float32)
    idxs, dvs = lax.fori_loop(0, k, jbody, (idxs0, dvs0))
    idx_ref[:, :] = idxs
    d_ref[:, :] = dvs


def _knn(query, ref, k):
    m, n = query.shape[0], ref.shape[0]
    rr = jnp.sum(ref * ref, axis=1, keepdims=True)
    zpad_q = jnp.zeros((m, 5), jnp.float32)
    zpad_r = jnp.zeros((n, 4), jnp.float32)
    A = jnp.concatenate([query, zpad_q], axis=1)
    B = jnp.concatenate([ref, rr, zpad_r], axis=1)
    BT = B.T
    kpad = 32 if k > 8 else 8
    BR = 128 if m >= 128 else m
    chunk = 512 if n >= 512 else n
    grid = m // BR
    idx, dv = pl.pallas_call(
        functools.partial(_topk_body, n, k, kpad, chunk),
        grid=(grid,),
        in_specs=[
            pl.BlockSpec((BR, 8), lambda i: (i, 0)),
            pl.BlockSpec((8, n), lambda i: (0, 0)),
        ],
        out_specs=[
            pl.BlockSpec((BR, kpad), lambda i: (i, 0)),
            pl.BlockSpec((BR, kpad), lambda i: (i, 0)),
        ],
        out_shape=[
            jax.ShapeDtypeStruct((m, kpad), jnp.int32),
            jax.ShapeDtypeStruct((m, kpad), jnp.float32),
        ],
        scratch_shapes=[pltpu.VMEM((BR, n), jnp.float32)],
        compiler_params=pltpu.CompilerParams(
            dimension_semantics=("arbitrary",)),
    )(A, BT)
    return idx[:, :k], dv[:, :k]


def _bn(x):
    axes = tuple(range(x.ndim - 1))
    mean = jnp.mean(x, axis=axes, keepdims=True)
    var = jnp.var(x, axis=axes, keepdims=True)
    return (x - mean) / jnp.sqrt(var + 1e-5)


def _mlp(x, layers):
    for W, b in layers:
        x = jax.nn.relu(_bn(x @ W + b))
    return x


def kernel(pos, feat, offset, params):
    del offset
    feat0 = jnp.concatenate([pos, feat], axis=1)

    o = _fps(pos, 4096)
    pos_l = [pos, pos[o[:4096]], pos[o[:1024]], pos[o[:256]], pos[o[:64]]]

    feats = [feat0]
    for lvl, (name, nsub) in enumerate(
            [('sa1', 4096), ('sa2', 1024), ('sa3', 256), ('sa4', 64)]):
        p_in, p_out = pos_l[lvl], pos_l[lvl + 1]
        nn_idx, _ = _knn(p_out, p_in, 32)
        grouped_pos = p_in[nn_idx] - p_out[:, None, :]
        grouped = jnp.concatenate([grouped_pos, feats[-1][nn_idx]], axis=-1)
        feats.append(jnp.max(_mlp(grouped, params[name]), axis=1))

    f1, f2, f3, f4 = feats[1], feats[2], feats[3], feats[4]

    def fp(pos1, feat1, pos2, feat2, layers):
        idx, d = _knn(pos1, pos2, 3)
        w = 1.0 / (d + 1e-8)
        w = w / jnp.sum(w, axis=1, keepdims=True)
        interp = jnp.sum(feat2[idx] * w[..., None], axis=1)
        x = interp if feat1 is None else jnp.concatenate([feat1, interp], axis=-1)
        return _mlp(x, layers)

    f3 = fp(pos_l[3], f3, pos_l[4], f4, params['fp4'])
    f2 = fp(pos_l[2], f2, pos_l[3], f3, params['fp3'])
    f1 = fp(pos_l[1], f1, pos_l[2], f2, params['fp2'])
    x = fp(pos_l[0], None, pos_l[1], f1, params['fp1'])
    (w1, b1), (w2, b2) = params['cls']
    x = jax.nn.relu(_bn(x @ w1 + b1))
    x = x @ w2 + b2
    return x


# speedup vs baseline: 1.3603x; 1.3603x over previous
import functools

import jax
import jax.numpy as jnp
from jax import lax
from jax.experimental import pallas as pl
from jax.experimental.pallas import tpu as pltpu


def _fps_body(m, x_ref, y_ref, z_ref, out_ref):
    R = x_ref.shape[0]
    x = x_ref[:, :]
    y = y_ref[:, :]
    z = z_ref[:, :]
    rows_out = m // 128
    iota_flat = (lax.broadcasted_iota(jnp.int32, (R, 128), 0) * 128
                 + lax.broadcasted_iota(jnp.int32, (R, 128), 1))
    lane = lax.broadcasted_iota(jnp.int32, (1, 128), 1)
    out_iota = (lax.broadcasted_iota(jnp.int32, (rows_out, 128), 0) * 128
                + lax.broadcasted_iota(jnp.int32, (rows_out, 128), 1))

    def body(i, state):
        dists, idxs, r, c = state
        idxs = jnp.where(out_iota == i, r * 128 + c, idxs)
        cm = lane == c
        xf = jnp.sum(jnp.where(cm, x_ref[pl.ds(r, 1), :], 0.0))
        yf = jnp.sum(jnp.where(cm, y_ref[pl.ds(r, 1), :], 0.0))
        zf = jnp.sum(jnp.where(cm, z_ref[pl.ds(r, 1), :], 0.0))
        d = (x - xf) ** 2 + (y - yf) ** 2 + (z - zf) ** 2
        dists = jnp.minimum(dists, d)
        mx = jnp.max(dists)
        far = jnp.min(jnp.where(dists == mx, iota_flat, jnp.int32(R * 128)))
        return dists, idxs, far // 128, far % 128

    dists0 = jnp.full((R, 128), 1e10, dtype=jnp.float32)
    idxs0 = jnp.zeros((rows_out, 128), dtype=jnp.int32)
    _, idxs, _, _ = lax.fori_loop(
        0, m, body, (dists0, idxs0, jnp.int32(0), jnp.int32(0)))
    out_ref[:, :] = idxs


def _fps(pos, m):
    n = pos.shape[0]
    x = pos[:, 0].reshape(n // 128, 128)
    y = pos[:, 1].reshape(n // 128, 128)
    z = pos[:, 2].reshape(n // 128, 128)
    idxs = pl.pallas_call(
        functools.partial(_fps_body, m),
        out_shape=jax.ShapeDtypeStruct((m // 128, 128), jnp.int32),
    )(x, y, z)
    return idxs.reshape(m)


def _sel3_body(n, chunk, d_ref, idx_ref, dv_ref):
    BR = d_ref.shape[0]
    nchunks = n // chunk
    BIG = jnp.float32(3.0e38)

    def cbody(c, st):
        mn1, am1, mn2, am2, mn3, am3 = st
        base = c * chunk
        v = d_ref[:, pl.ds(base, chunk)]
        ci = lax.broadcasted_iota(jnp.int32, (BR, chunk), 1) + base
        for _ in range(3):
            cm = jnp.min(v, axis=1, keepdims=True)
            ca = jnp.min(jnp.where(v == cm, ci, jnp.int32(n)), axis=1,
                         keepdims=True)
            v = jnp.where(ci == ca, BIG, v)
            c1 = cm < mn1
            c2 = cm < mn2
            c3 = cm < mn3
            nm1 = jnp.where(c1, cm, mn1)
            na1 = jnp.where(c1, ca, am1)
            nm2 = jnp.where(c1, mn1, jnp.where(c2, cm, mn2))
            na2 = jnp.where(c1, am1, jnp.where(c2, ca, am2))
            nm3 = jnp.where(c2, mn2, jnp.where(c3, cm, mn3))
            na3 = jnp.where(c2, am2, jnp.where(c3, ca, am3))
            mn1, am1, mn2, am2, mn3, am3 = nm1, na1, nm2, na2, nm3, na3
        return mn1, am1, mn2, am2, mn3, am3

    f0 = jnp.full((BR, 1), BIG, jnp.float32)
    i0 = jnp.full((BR, 1), n, jnp.int32)
    mn1, am1, mn2, am2, mn3, am3 = lax.fori_loop(
        0, nchunks, cbody, (f0, i0, f0, i0, f0, i0))
    lane = lax.broadcasted_iota(jnp.int32, (BR, 8), 1)
    idx_ref[:, :] = jnp.where(lane == 0, am1,
                              jnp.where(lane == 1, am2,
                                        jnp.where(lane == 2, am3, 0)))
    dv_ref[:, :] = jnp.where(lane == 0, mn1,
                             jnp.where(lane == 1, mn2,
                                       jnp.where(lane == 2, mn3, 0.0)))


def _knn(query, ref, k):
    m, n = query.shape[0], ref.shape[0]
    d = (jnp.sum(query * query, axis=1, keepdims=True)
         + jnp.sum(ref * ref, axis=1)[None, :]
         - 2.0 * (query @ ref.T))
    if k != 3:
        neg_d, idx = lax.top_k(-d, k)
        return idx, -neg_d
    BR = 128 if m >= 128 else m
    chunk = 512 if n >= 512 else n
    idx, dv = pl.pallas_call(
        functools.partial(_sel3_body, n, chunk),
        grid=(m // BR,),
        in_specs=[pl.BlockSpec((BR, n), lambda i: (i, 0))],
        out_specs=[
            pl.BlockSpec((BR, 8), lambda i: (i, 0)),
            pl.BlockSpec((BR, 8), lambda i: (i, 0)),
        ],
        out_shape=[
            jax.ShapeDtypeStruct((m, 8), jnp.int32),
            jax.ShapeDtypeStruct((m, 8), jnp.float32),
        ],
        compiler_params=pltpu.CompilerParams(
            dimension_semantics=("arbitrary",)),
    )(d)
    return idx[:, :3], dv[:, :3]


def _bn(x):
    axes = tuple(range(x.ndim - 1))
    mean = jnp.mean(x, axis=axes, keepdims=True)
    var = jnp.var(x, axis=axes, keepdims=True)
    return (x - mean) / jnp.sqrt(var + 1e-5)


def _mlp(x, layers):
    for W, b in layers:
        x = jax.nn.relu(_bn(x @ W + b))
    return x


def kernel(pos, feat, offset, params):
    del offset
    feat0 = jnp.concatenate([pos, feat], axis=1)

    o = _fps(pos, 4096)
    pos_l = [pos, pos[o[:4096]], pos[o[:1024]], pos[o[:256]], pos[o[:64]]]

    feats = [feat0]
    for lvl, (name, nsub) in enumerate(
            [('sa1', 4096), ('sa2', 1024), ('sa3', 256), ('sa4', 64)]):
        p_in, p_out = pos_l[lvl], pos_l[lvl + 1]
        nn_idx, _ = _knn(p_out, p_in, 32)
        grouped_pos = p_in[nn_idx] - p_out[:, None, :]
        grouped = jnp.concatenate([grouped_pos, feats[-1][nn_idx]], axis=-1)
        feats.append(jnp.max(_mlp(grouped, params[name]), axis=1))

    f1, f2, f3, f4 = feats[1], feats[2], feats[3], feats[4]

    def fp(pos1, feat1, pos2, feat2, layers):
        idx, d = _knn(pos1, pos2, 3)
        w = 1.0 / (d + 1e-8)
        w = w / jnp.sum(w, axis=1, keepdims=True)
        interp = jnp.sum(feat2[idx] * w[..., None], axis=1)
        x = interp if feat1 is None else jnp.concatenate([feat1, interp], axis=-1)
        return _mlp(x, layers)

    f3 = fp(pos_l[3], f3, pos_l[4], f4, params['fp4'])
    f2 = fp(pos_l[2], f2, pos_l[3], f3, params['fp3'])
    f1 = fp(pos_l[1], f1, pos_l[2], f2, params['fp2'])
    x = fp(pos_l[0], None, pos_l[1], f1, params['fp1'])
    (w1, b1), (w2, b2) = params['cls']
    x = jax.nn.relu(_bn(x @ w1 + b1))
    x = x @ w2 + b2
    return x
